# baseline (speedup 1.0000x reference)
import functools

import jax
import jax.numpy as jnp
from jax.experimental import pallas as pl
from jax.experimental.pallas import tpu as pltpu

_N = 4096
_M = 2048
_NH = 256

_BN = 512
_GI = _N // _BN
_GJ = _N // _BN

_HIGHEST = jax.lax.Precision.HIGHEST
_HIGH = jax.lax.Precision.HIGH


def _mm_kernel(x_ref, w_ref, o_ref):
    o_ref[...] = jnp.dot(x_ref[...], w_ref[...],
                         preferred_element_type=jnp.float32,
                         precision=_HIGHEST)


def _small_matmul(x, w):
    n, k = x.shape
    k2, m = w.shape
    return pl.pallas_call(
        _mm_kernel,
        grid=(n // _BN,),
        in_specs=[
            pl.BlockSpec((_BN, k), lambda i: (i, 0)),
            pl.BlockSpec((k2, m), lambda i: (0, 0)),
        ],
        out_specs=pl.BlockSpec((_BN, m), lambda i: (i, 0)),
        out_shape=jax.ShapeDtypeStruct((n, m), jnp.float32),
    )(x, w)


def _l1_kernel(wv_ref, adj_ref, adjv_ref, adje_ref, xw_ref, b1_ref,
               a_ref, h1_ref):
    j = pl.program_id(1)
    cv = wv_ref[0, 0]
    ce = 2.0 - cv
    a_tile = adj_ref[...] * (cv * adjv_ref[...] + ce * adje_ref[...])
    a_ref[...] = a_tile
    contrib = jnp.dot(a_tile, xw_ref[...],
                      preferred_element_type=jnp.float32,
                      precision=_HIGHEST)

    @pl.when(j == 0)
    def _():
        h1_ref[...] = contrib

    @pl.when(j > 0)
    def _():
        h1_ref[...] += contrib

    @pl.when(j == _GJ - 1)
    def _():
        h1_ref[...] = jax.nn.relu(h1_ref[...] + 2.0 * b1_ref[...])


def _layer1(wv2d, adj, adj_v, adj_e, xw, b1row):
    return pl.pallas_call(
        _l1_kernel,
        grid=(_GI, _GJ),
        in_specs=[
            pl.BlockSpec((1, 1), lambda i, j: (0, 0)),
            pl.BlockSpec((_BN, _BN), lambda i, j: (i, j)),
            pl.BlockSpec((_BN, _BN), lambda i, j: (i, j)),
            pl.BlockSpec((_BN, _BN), lambda i, j: (i, j)),
            pl.BlockSpec((_BN, _NH), lambda i, j: (j, 0)),
            pl.BlockSpec((1, _NH), lambda i, j: (0, 0)),
        ],
        out_specs=[
            pl.BlockSpec((_BN, _BN), lambda i, j: (i, j)),
            pl.BlockSpec((_BN, _NH), lambda i, j: (i, 0)),
        ],
        out_shape=[
            jax.ShapeDtypeStruct((_N, _N), jnp.float32),
            jax.ShapeDtypeStruct((_N, _NH), jnp.float32),
        ],
    )(wv2d, adj, adj_v, adj_e, xw, b1row)


def _l2_kernel(a_ref, hw_ref, b2_ref, wi_ref, v_ref, acc_ref):
    j = pl.program_id(1)
    contrib = jnp.dot(a_ref[...], hw_ref[...],
                      preferred_element_type=jnp.float32,
                      precision=_HIGHEST)

    @pl.when(j == 0)
    def _():
        acc_ref[...] = contrib

    @pl.when(j > 0)
    def _():
        acc_ref[...] += contrib

    @pl.when(j == _GJ - 1)
    def _():
        h2 = jax.nn.relu(acc_ref[...] + 2.0 * b2_ref[...])
        v_ref[...] = jnp.dot(h2, wi_ref[...],
                             preferred_element_type=jnp.float32,
                             precision=_HIGHEST)


def _layer2(a_mat, hw, b2row, wi):
    return pl.pallas_call(
        _l2_kernel,
        grid=(_GI, _GJ),
        in_specs=[
            pl.BlockSpec((_BN, _BN), lambda i, j: (i, j)),
            pl.BlockSpec((_BN, _NH), lambda i, j: (j, 0)),
            pl.BlockSpec((1, _NH), lambda i, j: (0, 0)),
            pl.BlockSpec((_NH, 1), lambda i, j: (0, 0)),
        ],
        out_specs=pl.BlockSpec((_BN, 1), lambda i, j: (i, 0)),
        out_shape=jax.ShapeDtypeStruct((_N, 1), jnp.float32),
        scratch_shapes=[pltpu.VMEM((_BN, _NH), jnp.float32)],
    )(a_mat, hw, b2row, wi)


def _final_kernel(pet_ref, v_ref, bi_ref, o_ref):
    h3 = jnp.dot(pet_ref[...], v_ref[...],
                 preferred_element_type=jnp.float32,
                 precision=_HIGHEST) + bi_ref[0, 0]
    o_ref[...] = jax.nn.sigmoid(h3)


def _final(pet, v, bi2d):
    bm = 512
    return pl.pallas_call(
        _final_kernel,
        grid=(_M // bm,),
        in_specs=[
            pl.BlockSpec((bm, _N), lambda i: (i, 0)),
            pl.BlockSpec((_N, 1), lambda i: (0, 0)),
            pl.BlockSpec((1, 1), lambda i: (0, 0)),
        ],
        out_specs=pl.BlockSpec((bm, 1), lambda i: (i, 0)),
        out_shape=jax.ShapeDtypeStruct((_M, 1), jnp.float32),
    )(pet, v, bi2d)


@jax.jit
def kernel(x, adj, adj_v, adj_e, PeT, wv, W1, b1, W2, b2, Wi, bi):
    wv2d = wv.reshape(1, 1).astype(jnp.float32)
    b1row = b1.reshape(1, _NH)
    b2row = b2.reshape(1, _NH)
    bi2d = bi.reshape(1, 1)

    xw = _small_matmul(x, W1)
    a_mat, h1 = _layer1(wv2d, adj, adj_v, adj_e, xw, b1row)
    hw = _small_matmul(h1, W2)
    v = _layer2(a_mat, hw, b2row, Wi)
    return _final(PeT, v, bi2d)


# speedup vs baseline: 1.3581x; 1.3581x over previous
import jax
import jax.numpy as jnp
from jax.experimental import pallas as pl
from jax.experimental.pallas import tpu as pltpu

_N = 4096
_M = 2048
_NH = 256

_BN = 512
_GI = _N // _BN
_GJ = _N // _BN

_HIGHEST = jax.lax.Precision.HIGHEST


def _mm_kernel(x_ref, w_ref, o_ref):
    o_ref[...] = jnp.dot(
        x_ref[...].astype(jnp.bfloat16), w_ref[...].astype(jnp.bfloat16),
        preferred_element_type=jnp.float32).astype(jnp.bfloat16)


def _small_matmul(x, w):
    n, k = x.shape
    k2, m = w.shape
    return pl.pallas_call(
        _mm_kernel,
        grid=(n // _BN,),
        in_specs=[
            pl.BlockSpec((_BN, k), lambda i: (i, 0)),
            pl.BlockSpec((k2, m), lambda i: (0, 0)),
        ],
        out_specs=pl.BlockSpec((_BN, m), lambda i: (i, 0)),
        out_shape=jax.ShapeDtypeStruct((n, m), jnp.bfloat16),
    )(x, w)


def _l1_kernel(wv_ref, adj_ref, adjv_ref, adje_ref, xw_ref, b1_ref,
               a_ref, h1_ref, acc_ref):
    j = pl.program_id(1)
    cv = wv_ref[0, 0]
    ce = 2.0 - cv
    a_bf = (adj_ref[...] * (cv * adjv_ref[...] + ce * adje_ref[...])
            ).astype(jnp.bfloat16)
    a_ref[...] = a_bf
    contrib = jnp.dot(a_bf, xw_ref[...], preferred_element_type=jnp.float32)

    @pl.when(j == 0)
    def _():
        acc_ref[...] = contrib

    @pl.when(j > 0)
    def _():
        acc_ref[...] += contrib

    @pl.when(j == _GJ - 1)
    def _():
        h1_ref[...] = jax.nn.relu(
            acc_ref[...] + 2.0 * b1_ref[...]).astype(jnp.bfloat16)


def _layer1(wv2d, adj, adj_v, adj_e, xw, b1row):
    return pl.pallas_call(
        _l1_kernel,
        grid=(_GI, _GJ),
        in_specs=[
            pl.BlockSpec((1, 1), lambda i, j: (0, 0)),
            pl.BlockSpec((_BN, _BN), lambda i, j: (i, j)),
            pl.BlockSpec((_BN, _BN), lambda i, j: (i, j)),
            pl.BlockSpec((_BN, _BN), lambda i, j: (i, j)),
            pl.BlockSpec((_BN, _NH), lambda i, j: (j, 0)),
            pl.BlockSpec((1, _NH), lambda i, j: (0, 0)),
        ],
        out_specs=[
            pl.BlockSpec((_BN, _BN), lambda i, j: (i, j)),
            pl.BlockSpec((_BN, _NH), lambda i, j: (i, 0)),
        ],
        out_shape=[
            jax.ShapeDtypeStruct((_N, _N), jnp.bfloat16),
            jax.ShapeDtypeStruct((_N, _NH), jnp.bfloat16),
        ],
        scratch_shapes=[pltpu.VMEM((_BN, _NH), jnp.float32)],
    )(wv2d, adj, adj_v, adj_e, xw, b1row)


def _l2_kernel(a_ref, hw_ref, b2_ref, wi_ref, pet_ref, bi_ref,
               o_ref, acc_ref):
    i = pl.program_id(0)
    j = pl.program_id(1)
    contrib = jnp.dot(a_ref[...], hw_ref[...],
                      preferred_element_type=jnp.float32)

    @pl.when(j == 0)
    def _():
        acc_ref[...] = contrib

    @pl.when(j > 0)
    def _():
        acc_ref[...] += contrib

    @pl.when(j == _GJ - 1)
    def _():
        h2 = jax.nn.relu(acc_ref[...] + 2.0 * b2_ref[...])
        v_i = jnp.dot(h2, wi_ref[...],
                      preferred_element_type=jnp.float32,
                      precision=_HIGHEST)
        o_contrib = jnp.dot(pet_ref[...], v_i,
                            preferred_element_type=jnp.float32,
                            precision=_HIGHEST)

        @pl.when(i == 0)
        def _():
            o_ref[...] = o_contrib

        @pl.when(i > 0)
        def _():
            o_ref[...] += o_contrib

        @pl.when(i == _GI - 1)
        def _():
            o_ref[...] = jax.nn.sigmoid(o_ref[...] + bi_ref[0, 0])


def _layer2(a_mat, hw, b2row, wi, pet, bi2d):
    return pl.pallas_call(
        _l2_kernel,
        grid=(_GI, _GJ),
        in_specs=[
            pl.BlockSpec((_BN, _BN), lambda i, j: (i, j)),
            pl.BlockSpec((_BN, _NH), lambda i, j: (j, 0)),
            pl.BlockSpec((1, _NH), lambda i, j: (0, 0)),
            pl.BlockSpec((_NH, 1), lambda i, j: (0, 0)),
            pl.BlockSpec((_M, _BN), lambda i, j: (0, i)),
            pl.BlockSpec((1, 1), lambda i, j: (0, 0)),
        ],
        out_specs=pl.BlockSpec((_M, 1), lambda i, j: (0, 0)),
        out_shape=jax.ShapeDtypeStruct((_M, 1), jnp.float32),
        scratch_shapes=[pltpu.VMEM((_BN, _NH), jnp.float32)],
    )(a_mat, hw, b2row, wi, pet, bi2d)


@jax.jit
def kernel(x, adj, adj_v, adj_e, PeT, wv, W1, b1, W2, b2, Wi, bi):
    wv2d = wv.reshape(1, 1).astype(jnp.float32)
    b1row = b1.reshape(1, _NH)
    b2row = b2.reshape(1, _NH)
    bi2d = bi.reshape(1, 1)

    xw = _small_matmul(x, W1)
    a_mat, h1 = _layer1(wv2d, adj, adj_v, adj_e, xw, b1row)
    hw = _small_matmul(h1, W2)
    return _layer2(a_mat, hw, b2row, Wi, PeT, bi2d)


# speedup vs baseline: 1.8636x; 1.3722x over previous
import jax
import jax.numpy as jnp
from jax.experimental import pallas as pl
from jax.experimental.pallas import tpu as pltpu

_N = 4096
_M = 2048
_NH = 256

_BN = 512
_GI = _N // _BN
_GJ = _N // _BN

_HIGHEST = jax.lax.Precision.HIGHEST


def _mm_kernel(x_ref, w_ref, o_ref):
    o_ref[...] = jnp.dot(
        x_ref[...].astype(jnp.bfloat16), w_ref[...].astype(jnp.bfloat16),
        preferred_element_type=jnp.float32).astype(jnp.bfloat16)


def _small_matmul(x, w):
    n, k = x.shape
    k2, m = w.shape
    return pl.pallas_call(
        _mm_kernel,
        grid=(n // _BN,),
        in_specs=[
            pl.BlockSpec((_BN, k), lambda i: (i, 0)),
            pl.BlockSpec((k2, m), lambda i: (0, 0)),
        ],
        out_specs=pl.BlockSpec((_BN, m), lambda i: (i, 0)),
        out_shape=jax.ShapeDtypeStruct((n, m), jnp.bfloat16),
    )(x, w)


def _mega_kernel(wv_ref, adj_ref, adjv_ref, adje_ref, xw_ref, b1_ref,
                 w2_ref, b2_ref, wi_ref, pet_ref, bi_ref,
                 o_ref,
                 a_vmem, h1_vmem, hw_vmem, acc_ref):
    p = pl.program_id(0)
    i = pl.program_id(1)
    j = pl.program_id(2)

    @pl.when(p == 0)
    def _phase0():
        cv = wv_ref[0, 0]
        ce = 2.0 - cv
        a_bf = (adj_ref[...] * (cv * adjv_ref[...] + ce * adje_ref[...])
                ).astype(jnp.bfloat16)
        a_vmem[i, j] = a_bf
        contrib = jnp.dot(a_bf, xw_ref[...],
                          preferred_element_type=jnp.float32)

        @pl.when(j == 0)
        def _():
            acc_ref[...] = contrib

        @pl.when(j > 0)
        def _():
            acc_ref[...] += contrib

        @pl.when(j == _GJ - 1)
        def _():
            h1_vmem[i] = jax.nn.relu(
                acc_ref[...] + 2.0 * b1_ref[...]).astype(jnp.bfloat16)

    @pl.when(p == 1)
    def _phase1():
        @pl.when((i == 0) & (j == 0))
        def _():
            w2_bf = w2_ref[...].astype(jnp.bfloat16)
            for jb in range(_GJ):
                hw_vmem[jb] = jnp.dot(
                    h1_vmem[jb], w2_bf,
                    preferred_element_type=jnp.float32).astype(jnp.bfloat16)

        contrib = jnp.dot(a_vmem[i, j], hw_vmem[j],
                          preferred_element_type=jnp.float32)

        @pl.when(j == 0)
        def _():
            acc_ref[...] = contrib

        @pl.when(j > 0)
        def _():
            acc_ref[...] += contrib

        @pl.when(j == _GJ - 1)
        def _():
            h2 = jax.nn.relu(acc_ref[...] + 2.0 * b2_ref[...])
            v_i = jnp.dot(h2, wi_ref[...],
                          preferred_element_type=jnp.float32,
                          precision=_HIGHEST)
            o_contrib = jnp.dot(pet_ref[...].astype(jnp.bfloat16),
                                v_i.astype(jnp.bfloat16),
                                preferred_element_type=jnp.float32)

            @pl.when(i == 0)
            def _():
                o_ref[...] = o_contrib

            @pl.when(i > 0)
            def _():
                o_ref[...] += o_contrib

            @pl.when(i == _GI - 1)
            def _():
                o_ref[...] = jax.nn.sigmoid(o_ref[...] + bi_ref[0, 0])


def _mega(wv2d, adj, adj_v, adj_e, xw, b1row, w2, b2row, wi, pet, bi2d):
    def tile_map(p, i, j):
        return (jnp.where(p == 0, i, _GI - 1), jnp.where(p == 0, j, _GJ - 1))

    def xw_map(p, i, j):
        return (jnp.where(p == 0, j, _GJ - 1), 0)

    def pet_map(p, i, j):
        return (0, jnp.where(p == 0, 0, i))

    zero2 = lambda p, i, j: (0, 0)

    return pl.pallas_call(
        _mega_kernel,
        grid=(2, _GI, _GJ),
        in_specs=[
            pl.BlockSpec((1, 1), zero2),
            pl.BlockSpec((_BN, _BN), tile_map),
            pl.BlockSpec((_BN, _BN), tile_map),
            pl.BlockSpec((_BN, _BN), tile_map),
            pl.BlockSpec((_BN, _NH), xw_map),
            pl.BlockSpec((1, _NH), zero2),
            pl.BlockSpec((_NH, _NH), zero2),
            pl.BlockSpec((1, _NH), zero2),
            pl.BlockSpec((_NH, 1), zero2),
            pl.BlockSpec((_M, _BN), pet_map),
            pl.BlockSpec((1, 1), zero2),
        ],
        out_specs=pl.BlockSpec((_M, 1), zero2),
        out_shape=jax.ShapeDtypeStruct((_M, 1), jnp.float32),
        scratch_shapes=[
            pltpu.VMEM((_GI, _GJ, _BN, _BN), jnp.bfloat16),
            pltpu.VMEM((_GI, _BN, _NH), jnp.bfloat16),
            pltpu.VMEM((_GJ, _BN, _NH), jnp.bfloat16),
            pltpu.VMEM((_BN, _NH), jnp.float32),
        ],
        compiler_params=pltpu.CompilerParams(
            vmem_limit_bytes=100 * 1024 * 1024,
        ),
    )(wv2d, adj, adj_v, adj_e, xw, b1row, w2, b2row, wi, pet, bi2d)


@jax.jit
def kernel(x, adj, adj_v, adj_e, PeT, wv, W1, b1, W2, b2, Wi, bi):
    wv2d = wv.reshape(1, 1).astype(jnp.float32)
    b1row = b1.reshape(1, _NH)
    b2row = b2.reshape(1, _NH)
    bi2d = bi.reshape(1, 1)

    xw = _small_matmul(x, W1)
    return _mega(wv2d, adj, adj_v, adj_e, xw, b1row, W2, b2row, Wi, PeT,
                 bi2d)


# speedup vs baseline: 2.1100x; 1.1322x over previous
import jax
import jax.numpy as jnp
from jax.experimental import pallas as pl
from jax.experimental.pallas import tpu as pltpu

_N = 4096
_M = 2048
_NH = 256

_BR = 128
_GR = _N // _BR
_GP = _M // _BR

_HIGHEST = jax.lax.Precision.HIGHEST


def _mm_kernel(x_ref, w_ref, o_ref):
    o_ref[...] = jnp.dot(
        x_ref[...].astype(jnp.bfloat16), w_ref[...].astype(jnp.bfloat16),
        preferred_element_type=jnp.float32).astype(jnp.bfloat16)


def _small_matmul(x, w):
    n, k = x.shape
    k2, m = w.shape
    return pl.pallas_call(
        _mm_kernel,
        grid=(n // 512,),
        in_specs=[
            pl.BlockSpec((512, k), lambda i: (i, 0)),
            pl.BlockSpec((k2, m), lambda i: (0, 0)),
        ],
        out_specs=pl.BlockSpec((512, m), lambda i: (i, 0)),
        out_shape=jax.ShapeDtypeStruct((n, m), jnp.bfloat16),
    )(x, w)


def _mega_kernel(wv_ref, adj_ref, adjv_ref, adje_ref, xw_ref, b1_ref,
                 w2_ref, b2_ref, wi_ref, pet_ref, bi_ref,
                 o_ref,
                 a_vmem, h1_vmem, hw_vmem, v_vmem):
    p = pl.program_id(0)
    i = pl.program_id(1)

    @pl.when(p == 0)
    def _phase0():
        cv = wv_ref[0, 0]
        ce = 2.0 - cv
        a_bf = (adj_ref[...] * (cv * adjv_ref[...] + ce * adje_ref[...])
                ).astype(jnp.bfloat16)
        a_vmem[i] = a_bf
        h1 = jnp.dot(a_bf, xw_ref[...], preferred_element_type=jnp.float32)
        h1_vmem[pl.ds(i * _BR, _BR), :] = jax.nn.relu(
            h1 + 2.0 * b1_ref[...]).astype(jnp.bfloat16)

    @pl.when(p == 1)
    def _phase1():
        @pl.when(i == 0)
        def _():
            w2_bf = w2_ref[...].astype(jnp.bfloat16)
            for jb in range(0, _GR, 4):
                hw_vmem[pl.ds(jb * _BR, 4 * _BR), :] = jnp.dot(
                    h1_vmem[pl.ds(jb * _BR, 4 * _BR), :], w2_bf,
                    preferred_element_type=jnp.float32).astype(jnp.bfloat16)

        h2 = jax.nn.relu(
            jnp.dot(a_vmem[i], hw_vmem[...],
                    preferred_element_type=jnp.float32)
            + 2.0 * b2_ref[...])
        v_vmem[pl.ds(i * _BR, _BR), :] = jnp.dot(
            h2, wi_ref[...], preferred_element_type=jnp.float32,
            precision=_HIGHEST)

    @pl.when((p == 2) & (i < _GP))
    def _phase2():
        h3 = jnp.dot(pet_ref[...].astype(jnp.bfloat16),
                     v_vmem[...].astype(jnp.bfloat16),
                     preferred_element_type=jnp.float32) + bi_ref[0, 0]
        o_ref[...] = jax.nn.sigmoid(h3)


def _mega(wv2d, adj, adj_v, adj_e, xw, b1row, w2, b2row, wi, pet, bi2d):
    def panel_map(p, i):
        return (jnp.where(p == 0, i, _GR - 1), 0)

    def pet_map(p, i):
        return (jnp.where(p == 2, jnp.minimum(i, _GP - 1), 0), 0)

    def out_map(p, i):
        return (jnp.where(p == 2, jnp.minimum(i, _GP - 1), 0), 0)

    zero2 = lambda p, i: (0, 0)

    return pl.pallas_call(
        _mega_kernel,
        grid=(3, _GR),
        in_specs=[
            pl.BlockSpec((1, 1), zero2),
            pl.BlockSpec((_BR, _N), panel_map),
            pl.BlockSpec((_BR, _N), panel_map),
            pl.BlockSpec((_BR, _N), panel_map),
            pl.BlockSpec((_N, _NH), zero2),
            pl.BlockSpec((1, _NH), zero2),
            pl.BlockSpec((_NH, _NH), zero2),
            pl.BlockSpec((1, _NH), zero2),
            pl.BlockSpec((_NH, 1), zero2),
            pl.BlockSpec((_BR, _N), pet_map),
            pl.BlockSpec((1, 1), zero2),
        ],
        out_specs=pl.BlockSpec((_BR, 1), out_map),
        out_shape=jax.ShapeDtypeStruct((_M, 1), jnp.float32),
        scratch_shapes=[
            pltpu.VMEM((_GR, _BR, _N), jnp.bfloat16),
            pltpu.VMEM((_N, _NH), jnp.bfloat16),
            pltpu.VMEM((_N, _NH), jnp.bfloat16),
            pltpu.VMEM((_N, 1), jnp.float32),
        ],
        compiler_params=pltpu.CompilerParams(
            vmem_limit_bytes=100 * 1024 * 1024,
        ),
    )(wv2d, adj, adj_v, adj_e, xw, b1row, w2, b2row, wi, pet, bi2d)


@jax.jit
def kernel(x, adj, adj_v, adj_e, PeT, wv, W1, b1, W2, b2, Wi, bi):
    wv2d = wv.reshape(1, 1).astype(jnp.float32)
    b1row = b1.reshape(1, _NH)
    b2row = b2.reshape(1, _NH)
    bi2d = bi.reshape(1, 1)

    xw = _small_matmul(x, W1)
    return _mega(wv2d, adj, adj_v, adj_e, xw, b1row, W2, b2row, Wi, PeT,
                 bi2d)


# speedup vs baseline: 2.3669x; 1.1218x over previous
import jax
import jax.numpy as jnp
from jax.experimental import pallas as pl
from jax.experimental.pallas import tpu as pltpu

_N = 4096
_M = 2048
_NH = 256

_BR = 128
_GR = _N // _BR
_BP = 256
_GP = _M // _BP


def _mm_kernel(x_ref, w_ref, o_ref):
    o_ref[...] = jnp.dot(
        x_ref[...].astype(jnp.bfloat16), w_ref[...].astype(jnp.bfloat16),
        preferred_element_type=jnp.float32).astype(jnp.bfloat16)


def _small_matmul(x, w):
    n, k = x.shape
    k2, m = w.shape
    return pl.pallas_call(
        _mm_kernel,
        grid=(n // 512,),
        in_specs=[
            pl.BlockSpec((512, k), lambda i: (i, 0)),
            pl.BlockSpec((k2, m), lambda i: (0, 0)),
        ],
        out_specs=pl.BlockSpec((512, m), lambda i: (i, 0)),
        out_shape=jax.ShapeDtypeStruct((n, m), jnp.bfloat16),
    )(x, w)


def _mega_kernel(wv_ref, adj_ref, adjv_ref, adje_ref, xw_ref, b1_ref,
                 w2_ref, b2_ref, wi_ref, pet_ref, bi_ref,
                 o_ref,
                 a_vmem, h1_vmem, hw_vmem, v_vmem):
    p = pl.program_id(0)
    i = pl.program_id(1)

    @pl.when(p == 0)
    def _phase0():
        cv = wv_ref[0, 0]
        ce = 2.0 - cv
        a_bf = (adj_ref[...] * (cv * adjv_ref[...] + ce * adje_ref[...])
                ).astype(jnp.bfloat16)
        a_vmem[i] = a_bf
        h1 = jnp.dot(a_bf, xw_ref[...], preferred_element_type=jnp.float32)
        h1_vmem[pl.ds(i * _BR, _BR), :] = jax.nn.relu(
            h1 + 2.0 * b1_ref[...]).astype(jnp.bfloat16)

    @pl.when(p == 1)
    def _phase1():
        @pl.when(i == 0)
        def _():
            w2_bf = w2_ref[...].astype(jnp.bfloat16)
            for jb in range(0, _GR, 4):
                hw_vmem[pl.ds(jb * _BR, 4 * _BR), :] = jnp.dot(
                    h1_vmem[pl.ds(jb * _BR, 4 * _BR), :], w2_bf,
                    preferred_element_type=jnp.float32).astype(jnp.bfloat16)

        h2 = jax.nn.relu(
            jnp.dot(a_vmem[i], hw_vmem[...],
                    preferred_element_type=jnp.float32)
            + 2.0 * b2_ref[...])
        v_vmem[pl.ds(i * _BR, _BR), :] = jnp.dot(
            h2.astype(jnp.bfloat16), wi_ref[...].astype(jnp.bfloat16),
            preferred_element_type=jnp.float32)

    @pl.when((p == 2) & (i < _GP))
    def _phase2():
        h3 = jnp.dot(pet_ref[...].astype(jnp.bfloat16),
                     v_vmem[...].astype(jnp.bfloat16),
                     preferred_element_type=jnp.float32) + bi_ref[0, 0]
        o_ref[...] = jax.nn.sigmoid(h3)


def _mega(wv2d, adj, adj_v, adj_e, xw, b1row, w2, b2row, wi, pet, bi2d):
    def panel_map(p, i):
        return (jnp.where(p == 0, i, _GR - 1), 0)

    def pet_map(p, i):
        return (jnp.where(p == 2, jnp.minimum(i, _GP - 1), 0), 0)

    def out_map(p, i):
        return (jnp.where(p == 2, jnp.minimum(i, _GP - 1), 0), 0)

    zero2 = lambda p, i: (0, 0)

    return pl.pallas_call(
        _mega_kernel,
        grid=(3, _GR),
        in_specs=[
            pl.BlockSpec((1, 1), zero2),
            pl.BlockSpec((_BR, _N), panel_map),
            pl.BlockSpec((_BR, _N), panel_map),
            pl.BlockSpec((_BR, _N), panel_map),
            pl.BlockSpec((_N, _NH), zero2),
            pl.BlockSpec((1, _NH), zero2),
            pl.BlockSpec((_NH, _NH), zero2),
            pl.BlockSpec((1, _NH), zero2),
            pl.BlockSpec((_NH, 1), zero2),
            pl.BlockSpec((_BP, _N), pet_map),
            pl.BlockSpec((1, 1), zero2),
        ],
        out_specs=pl.BlockSpec((_BP, 1), out_map),
        out_shape=jax.ShapeDtypeStruct((_M, 1), jnp.float32),
        scratch_shapes=[
            pltpu.VMEM((_GR, _BR, _N), jnp.bfloat16),
            pltpu.VMEM((_N, _NH), jnp.bfloat16),
            pltpu.VMEM((_N, _NH), jnp.bfloat16),
            pltpu.VMEM((_N, 1), jnp.float32),
        ],
        compiler_params=pltpu.CompilerParams(
            vmem_limit_bytes=100 * 1024 * 1024,
        ),
    )(wv2d, adj, adj_v, adj_e, xw, b1row, w2, b2row, wi, pet, bi2d)


@jax.jit
def kernel(x, adj, adj_v, adj_e, PeT, wv, W1, b1, W2, b2, Wi, bi):
    wv2d = wv.reshape(1, 1).astype(jnp.float32)
    b1row = b1.reshape(1, _NH)
    b2row = b2.reshape(1, _NH)
    bi2d = bi.reshape(1, 1)

    xw = _small_matmul(x, W1)
    return _mega(wv2d, adj, adj_v, adj_e, xw, b1row, W2, b2row, Wi, PeT,
                 bi2d)
